# baseline (speedup 1.0000x reference)
import functools

import jax
import jax.numpy as jnp
from jax import lax
from jax.experimental import pallas as pl
from jax.experimental.pallas import tpu as pltpu
from jax.experimental.pallas import tpu_sc as plsc

_B = 16384
_F = 16
_HID = 64
_NW = 32
_ROWS_PER_W = _B // _NW
_CHUNK = 128
_NCHUNK = _ROWS_PER_W // _CHUNK
_L = 16

_NU = 1000000
_NM = 100000
_CU = _NU // 128
_CM = _NM // 128
_RU = _NU // 8
_RM_PAD = 12504

_SCALE = 5.0 - 0.5 + 1.0
_SHIFT = 0.5 - 0.5


def _permute_block(in_blk, out_blk):
    lanes = jax.lax.iota(jnp.int32, _L)
    for r in range(16):
        for s in range(8):
            out_blk[r, pl.ds(s * _L, _L)] = plsc.load_gather(
                in_blk, [lanes, jnp.full((_L,), r * 8 + s, jnp.int32)])


def _retile_table(tabt, rows_out, nfull, wid, ring, obuf, in_sem, out_sem):
    n_w = (nfull - wid + _NW - 1) // _NW

    def fire_in(t, slot):
        c = (wid + _NW * t) * 128
        pltpu.async_copy(tabt.at[:, pl.ds(c, 128)], ring.at[slot], in_sem)

    for t in range(4):
        fire_in(t, t)

    def body(t, _):
        slot = lax.rem(t, 4)
        ob = lax.rem(t, 2)
        pltpu.make_async_copy(tabt.at[:, pl.ds(0, 128)],
                              ring.at[slot], in_sem).wait()

        @pl.when(t >= 2)
        def _():
            pltpu.make_async_copy(obuf.at[ob],
                                  rows_out.at[pl.ds(0, 16), :],
                                  out_sem).wait()
        _permute_block(ring.at[slot], obuf.at[ob])
        c16 = (wid + _NW * t) * 16
        pltpu.async_copy(obuf.at[ob], rows_out.at[pl.ds(c16, 16), :], out_sem)

        @pl.when(t + 4 < n_w)
        def _():
            fire_in(t + 4, slot)
        return 0

    jax.lax.fori_loop(0, n_w, body, 0)
    for _ in range(2):
        pltpu.make_async_copy(obuf.at[0], rows_out.at[pl.ds(0, 16), :],
                              out_sem).wait()


def _retile_body(u_tabt, m_tabt, u_rows, m_rows,
                 ring, obuf, rem_in_u, rem_in_m, rem_out, in_sem, out_sem):
    wid = lax.axis_index("s") * 2 + lax.axis_index("c")
    _retile_table(u_tabt, u_rows, _CU, wid, ring, obuf, in_sem, out_sem)
    _retile_table(m_tabt, m_rows, _CM, wid, ring, obuf, in_sem, out_sem)

    lanes = jax.lax.iota(jnp.int32, _L)

    @pl.when(wid == 0)
    def _():
        pltpu.sync_copy(u_tabt.at[:, pl.ds(_CU * 128, 64)], rem_in_u)
        for r in range(8):
            for s in range(8):
                rem_out[r, pl.ds(s * _L, _L)] = plsc.load_gather(
                    rem_in_u, [lanes, jnp.full((_L,), r * 8 + s, jnp.int32)])
        pltpu.sync_copy(rem_out.at[pl.ds(0, 8), :],
                        u_rows.at[pl.ds(_CU * 16, 8), :])

    @pl.when(wid == 1)
    def _():
        pltpu.sync_copy(m_tabt.at[:, pl.ds(_CM * 128, 32)], rem_in_m)
        for r in range(8):
            for s in range(8):
                l = min(r * 8 + s, 31)
                rem_out[r, pl.ds(s * _L, _L)] = plsc.load_gather(
                    rem_in_m, [lanes, jnp.full((_L,), l, jnp.int32)])
        pltpu.sync_copy(rem_out.at[pl.ds(0, 8), :],
                        m_rows.at[pl.ds(_CM * 16, 8), :])


@functools.partial(
    pl.kernel,
    out_type=(
        jax.ShapeDtypeStruct((_RU, 128), jnp.float32),
        jax.ShapeDtypeStruct((_RM_PAD, 128), jnp.float32),
    ),
    mesh=plsc.VectorSubcoreMesh(core_axis_name="c", subcore_axis_name="s"),
    compiler_params=pltpu.CompilerParams(needs_layout_passes=False),
    scratch_types=[
        pltpu.VMEM((4, _F, 128), jnp.float32),
        pltpu.VMEM((2, _F, 128), jnp.float32),
        pltpu.VMEM((_F, 64), jnp.float32),
        pltpu.VMEM((_F, 32), jnp.float32),
        pltpu.VMEM((8, 128), jnp.float32),
        pltpu.SemaphoreType.DMA,
        pltpu.SemaphoreType.DMA,
    ],
)
def _retile(u_tabt, m_tabt, u_rows, m_rows,
            ring, obuf, rem_in_u, rem_in_m, rem_out, in_sem, out_sem):
    _retile_body(u_tabt, m_tabt, u_rows, m_rows,
                 ring, obuf, rem_in_u, rem_in_m, rem_out, in_sem, out_sem)


def _compute_tile_indices(idx_v, tidx_v):
    for j in range(_NCHUNK):
        def body(g, _, j=j):
            iv = idx_v[j, pl.ds(g * _L, _L)]
            tidx_v[j, pl.ds(g * _L, _L)] = jax.lax.shift_right_logical(iv, 3)
            return 0
        jax.lax.fori_loop(0, _CHUNK // _L, body, 0)


def _extract_chunk(idx_v, j, gat, stage):
    lanes = jax.lax.iota(jnp.int32, _L)

    def body(g, _):
        iv = idx_v[j, pl.ds(g * _L, _L)]
        sub = (iv & 7) * _F
        rows = g * _L + lanes
        off = j * _CHUNK + g * _L
        for c in range(_F):
            stage[c, pl.ds(off, _L)] = plsc.load_gather(gat, [rows, sub + c])
        return 0

    jax.lax.fori_loop(0, _CHUNK // _L, body, 0)


def _gather_body(u_idx, m_idx, u_tab, m_tab, u_out, m_out,
                 u_idx_v, m_idx_v, u_tidx, m_tidx, gat, u_stage, m_stage, sem):
    wid = lax.axis_index("s") * 2 + lax.axis_index("c")
    base = wid * _ROWS_PER_W
    pltpu.sync_copy(u_idx.at[pl.ds(wid * _NCHUNK, _NCHUNK)], u_idx_v)
    pltpu.sync_copy(m_idx.at[pl.ds(wid * _NCHUNK, _NCHUNK)], m_idx_v)
    _compute_tile_indices(u_idx_v, u_tidx)
    _compute_tile_indices(m_idx_v, m_tidx)

    tasks = [(u_tab, u_tidx, u_idx_v, u_stage, j) for j in range(_NCHUNK)]
    tasks += [(m_tab, m_tidx, m_idx_v, m_stage, j) for j in range(_NCHUNK)]

    def fire(t, buf):
        tab, tidx, _, _, j = tasks[t]
        return pltpu.async_copy(tab.at[tidx.at[j]], gat.at[buf], sem)

    handles = {0: fire(0, 0)}
    for t in range(len(tasks)):
        handles[t].wait()
        if t + 1 < len(tasks):
            handles[t + 1] = fire(t + 1, (t + 1) % 2)
        _, _, idx_v, stage, j = tasks[t]
        _extract_chunk(idx_v, j, gat.at[t % 2], stage)

    pltpu.sync_copy(u_stage, u_out.at[:, pl.ds(base, _ROWS_PER_W)])
    pltpu.sync_copy(m_stage, m_out.at[:, pl.ds(base, _ROWS_PER_W)])


@functools.partial(
    pl.kernel,
    out_type=(
        jax.ShapeDtypeStruct((_F, _B), jnp.float32),
        jax.ShapeDtypeStruct((_F, _B), jnp.float32),
    ),
    mesh=plsc.VectorSubcoreMesh(core_axis_name="c", subcore_axis_name="s"),
    compiler_params=pltpu.CompilerParams(needs_layout_passes=False),
    scratch_types=[
        pltpu.VMEM((_NCHUNK, _CHUNK), jnp.int32),
        pltpu.VMEM((_NCHUNK, _CHUNK), jnp.int32),
        pltpu.VMEM((_NCHUNK, _CHUNK), jnp.int32),
        pltpu.VMEM((_NCHUNK, _CHUNK), jnp.int32),
        pltpu.VMEM((2, _CHUNK, 128), jnp.float32),
        pltpu.VMEM((_F, _ROWS_PER_W), jnp.float32),
        pltpu.VMEM((_F, _ROWS_PER_W), jnp.float32),
        pltpu.SemaphoreType.DMA,
    ],
)
def _gather(u_idx, m_idx, u_tab, m_tab, u_out, m_out,
            u_idx_v, m_idx_v, u_tidx, m_tidx, gat, u_stage, m_stage, sem):
    _gather_body(u_idx, m_idx, u_tab, m_tab, u_out, m_out,
                 u_idx_v, m_idx_v, u_tidx, m_tidx, gat, u_stage, m_stage, sem)


_BLK = 2048


def _mlp_body(u_ref, m_ref, w1a_ref, w1b_ref, b1_ref, w2_ref, b2_ref, o_ref):
    h = jnp.dot(w1a_ref[...], u_ref[...],
                preferred_element_type=jnp.float32,
                precision=lax.Precision.HIGHEST)
    h = h + jnp.dot(w1b_ref[...], m_ref[...],
                    preferred_element_type=jnp.float32,
                    precision=lax.Precision.HIGHEST)
    h = jnp.maximum(h + b1_ref[...], 0.0)
    t = jnp.sum(h * w2_ref[...], axis=0, keepdims=True) + b2_ref[...]
    o_ref[...] = jax.nn.sigmoid(t) * _SCALE + _SHIFT


def _mlp(u_embt, m_embt, w1at, w1bt, b1, w2, b2):
    grid = (_B // _BLK,)
    return pl.pallas_call(
        _mlp_body,
        grid=grid,
        in_specs=[
            pl.BlockSpec((_F, _BLK), lambda i: (0, i)),
            pl.BlockSpec((_F, _BLK), lambda i: (0, i)),
            pl.BlockSpec((_HID, _F), lambda i: (0, 0)),
            pl.BlockSpec((_HID, _F), lambda i: (0, 0)),
            pl.BlockSpec((_HID, 1), lambda i: (0, 0)),
            pl.BlockSpec((_HID, 1), lambda i: (0, 0)),
            pl.BlockSpec((1, 1), lambda i: (0, 0)),
        ],
        out_specs=pl.BlockSpec((1, _BLK), lambda i: (0, i)),
        out_shape=jax.ShapeDtypeStruct((1, _B), jnp.float32),
    )(u_embt, m_embt, w1at, w1bt, b1, w2, b2)


def kernel(user, movie, u_table, m_table, W1, b1, W2, b2):
    u_rows, m_rows = _retile(u_table.T, m_table.T)
    u_idx = user.astype(jnp.int32).reshape(_NW * _NCHUNK, _CHUNK)
    m_idx = movie.astype(jnp.int32).reshape(_NW * _NCHUNK, _CHUNK)
    u_embt, m_embt = _gather(u_idx, m_idx, u_rows, m_rows)
    out = _mlp(u_embt, m_embt, W1[:_F].T, W1[_F:].T,
               b1.reshape(_HID, 1), W2.reshape(_HID, 1), b2.reshape(1, 1))
    return out.reshape(_B, 1)


# speedup vs baseline: 1.9098x; 1.9098x over previous
import functools

import jax
import jax.numpy as jnp
from jax import lax
from jax.experimental import pallas as pl
from jax.experimental.pallas import tpu as pltpu
from jax.experimental.pallas import tpu_sc as plsc

_B = 16384
_F = 16
_HID = 64
_NW = 32
_ROWS_PER_W = _B // _NW
_CHUNK = 128
_NCHUNK = _ROWS_PER_W // _CHUNK
_L = 16

_NU = 1000000
_NM = 100000
_CU = _NU // 128
_CM = _NM // 128
_RU = _NU // 8
_RM_PAD = 12504

_SCALE = 5.0 - 0.5 + 1.0
_SHIFT = 0.5 - 0.5


def _permute_block(in_blk, out_blk):
    lanes = jax.lax.iota(jnp.int32, _L)

    @plsc.parallel_loop(0, 128, unroll=8)
    def _(i):
        r = i >> 3
        s = i & 7
        out_blk[r, pl.ds(s * _L, _L)] = plsc.load_gather(
            in_blk, [lanes, lanes * 0 + i])


def _retile_table(tabt, rows_out, nfull, wid, ring, obuf, in_sem, out_sem):
    n_w = (nfull - wid + _NW - 1) // _NW

    def fire_in(t, slot):
        c = (wid + _NW * t) * 128
        pltpu.async_copy(tabt.at[:, pl.ds(c, 128)], ring.at[slot], in_sem)

    for t in range(4):
        fire_in(t, t)

    def body(t, _):
        slot = lax.rem(t, 4)
        ob = lax.rem(t, 2)
        pltpu.make_async_copy(tabt.at[:, pl.ds(0, 128)],
                              ring.at[slot], in_sem).wait()

        @pl.when(t >= 2)
        def _():
            pltpu.make_async_copy(obuf.at[ob],
                                  rows_out.at[pl.ds(0, 16), :],
                                  out_sem).wait()
        _permute_block(ring.at[slot], obuf.at[ob])
        c16 = (wid + _NW * t) * 16
        pltpu.async_copy(obuf.at[ob], rows_out.at[pl.ds(c16, 16), :], out_sem)

        @pl.when(t + 4 < n_w)
        def _():
            fire_in(t + 4, slot)
        return 0

    jax.lax.fori_loop(0, n_w, body, 0)
    for _ in range(2):
        pltpu.make_async_copy(obuf.at[0], rows_out.at[pl.ds(0, 16), :],
                              out_sem).wait()


def _retile_body(u_tabt, m_tabt, u_rows, m_rows,
                 ring, obuf, rem_in_u, rem_in_m, rem_out, in_sem, out_sem):
    wid = lax.axis_index("s") * 2 + lax.axis_index("c")
    _retile_table(u_tabt, u_rows, _CU, wid, ring, obuf, in_sem, out_sem)
    _retile_table(m_tabt, m_rows, _CM, wid, ring, obuf, in_sem, out_sem)

    lanes = jax.lax.iota(jnp.int32, _L)

    @pl.when(wid == 0)
    def _():
        pltpu.sync_copy(u_tabt.at[:, pl.ds(_CU * 128, 64)], rem_in_u)
        for r in range(8):
            for s in range(8):
                rem_out[r, pl.ds(s * _L, _L)] = plsc.load_gather(
                    rem_in_u, [lanes, jnp.full((_L,), r * 8 + s, jnp.int32)])
        pltpu.sync_copy(rem_out.at[pl.ds(0, 8), :],
                        u_rows.at[pl.ds(_CU * 16, 8), :])

    @pl.when(wid == 1)
    def _():
        pltpu.sync_copy(m_tabt.at[:, pl.ds(_CM * 128, 32)], rem_in_m)
        for r in range(8):
            for s in range(8):
                l = min(r * 8 + s, 31)
                rem_out[r, pl.ds(s * _L, _L)] = plsc.load_gather(
                    rem_in_m, [lanes, jnp.full((_L,), l, jnp.int32)])
        pltpu.sync_copy(rem_out.at[pl.ds(0, 8), :],
                        m_rows.at[pl.ds(_CM * 16, 8), :])


@functools.partial(
    pl.kernel,
    out_type=(
        jax.ShapeDtypeStruct((_RU, 128), jnp.float32),
        jax.ShapeDtypeStruct((_RM_PAD, 128), jnp.float32),
    ),
    mesh=plsc.VectorSubcoreMesh(core_axis_name="c", subcore_axis_name="s"),
    compiler_params=pltpu.CompilerParams(needs_layout_passes=False),
    scratch_types=[
        pltpu.VMEM((4, _F, 128), jnp.float32),
        pltpu.VMEM((2, _F, 128), jnp.float32),
        pltpu.VMEM((_F, 64), jnp.float32),
        pltpu.VMEM((_F, 32), jnp.float32),
        pltpu.VMEM((8, 128), jnp.float32),
        pltpu.SemaphoreType.DMA,
        pltpu.SemaphoreType.DMA,
    ],
)
def _retile(u_tabt, m_tabt, u_rows, m_rows,
            ring, obuf, rem_in_u, rem_in_m, rem_out, in_sem, out_sem):
    _retile_body(u_tabt, m_tabt, u_rows, m_rows,
                 ring, obuf, rem_in_u, rem_in_m, rem_out, in_sem, out_sem)


def _compute_tile_indices(idx_v, tidx_v):
    for j in range(_NCHUNK):
        def body(g, _, j=j):
            iv = idx_v[j, pl.ds(g * _L, _L)]
            tidx_v[j, pl.ds(g * _L, _L)] = jax.lax.shift_right_logical(iv, 3)
            return 0
        jax.lax.fori_loop(0, _CHUNK // _L, body, 0)


def _extract_chunk(idx_v, j, gat, stage):
    lanes = jax.lax.iota(jnp.int32, _L)

    def body(g, _):
        iv = idx_v[j, pl.ds(g * _L, _L)]
        sub = (iv & 7) * _F
        rows = g * _L + lanes
        off = j * _CHUNK + g * _L
        for c in range(_F):
            stage[c, pl.ds(off, _L)] = plsc.load_gather(gat, [rows, sub + c])
        return 0

    jax.lax.fori_loop(0, _CHUNK // _L, body, 0)


def _gather_body(u_idx, m_idx, u_tab, m_tab, u_out, m_out,
                 u_idx_v, m_idx_v, u_tidx, m_tidx, gat, u_stage, m_stage, sem):
    wid = lax.axis_index("s") * 2 + lax.axis_index("c")
    base = wid * _ROWS_PER_W
    pltpu.sync_copy(u_idx.at[pl.ds(wid * _NCHUNK, _NCHUNK)], u_idx_v)
    pltpu.sync_copy(m_idx.at[pl.ds(wid * _NCHUNK, _NCHUNK)], m_idx_v)
    _compute_tile_indices(u_idx_v, u_tidx)
    _compute_tile_indices(m_idx_v, m_tidx)

    tasks = [(u_tab, u_tidx, u_idx_v, u_stage, j) for j in range(_NCHUNK)]
    tasks += [(m_tab, m_tidx, m_idx_v, m_stage, j) for j in range(_NCHUNK)]

    def fire(t, buf):
        tab, tidx, _, _, j = tasks[t]
        return pltpu.async_copy(tab.at[tidx.at[j]], gat.at[buf], sem)

    handles = {0: fire(0, 0)}
    for t in range(len(tasks)):
        handles[t].wait()
        if t + 1 < len(tasks):
            handles[t + 1] = fire(t + 1, (t + 1) % 2)
        _, _, idx_v, stage, j = tasks[t]
        _extract_chunk(idx_v, j, gat.at[t % 2], stage)

    pltpu.sync_copy(u_stage, u_out.at[:, pl.ds(base, _ROWS_PER_W)])
    pltpu.sync_copy(m_stage, m_out.at[:, pl.ds(base, _ROWS_PER_W)])


@functools.partial(
    pl.kernel,
    out_type=(
        jax.ShapeDtypeStruct((_F, _B), jnp.float32),
        jax.ShapeDtypeStruct((_F, _B), jnp.float32),
    ),
    mesh=plsc.VectorSubcoreMesh(core_axis_name="c", subcore_axis_name="s"),
    compiler_params=pltpu.CompilerParams(needs_layout_passes=False),
    scratch_types=[
        pltpu.VMEM((_NCHUNK, _CHUNK), jnp.int32),
        pltpu.VMEM((_NCHUNK, _CHUNK), jnp.int32),
        pltpu.VMEM((_NCHUNK, _CHUNK), jnp.int32),
        pltpu.VMEM((_NCHUNK, _CHUNK), jnp.int32),
        pltpu.VMEM((2, _CHUNK, 128), jnp.float32),
        pltpu.VMEM((_F, _ROWS_PER_W), jnp.float32),
        pltpu.VMEM((_F, _ROWS_PER_W), jnp.float32),
        pltpu.SemaphoreType.DMA,
    ],
)
def _gather(u_idx, m_idx, u_tab, m_tab, u_out, m_out,
            u_idx_v, m_idx_v, u_tidx, m_tidx, gat, u_stage, m_stage, sem):
    _gather_body(u_idx, m_idx, u_tab, m_tab, u_out, m_out,
                 u_idx_v, m_idx_v, u_tidx, m_tidx, gat, u_stage, m_stage, sem)


_BLK = 2048


def _mlp_body(u_ref, m_ref, w1a_ref, w1b_ref, b1_ref, w2_ref, b2_ref, o_ref):
    h = jnp.dot(w1a_ref[...], u_ref[...],
                preferred_element_type=jnp.float32,
                precision=lax.Precision.HIGHEST)
    h = h + jnp.dot(w1b_ref[...], m_ref[...],
                    preferred_element_type=jnp.float32,
                    precision=lax.Precision.HIGHEST)
    h = jnp.maximum(h + b1_ref[...], 0.0)
    t = jnp.sum(h * w2_ref[...], axis=0, keepdims=True) + b2_ref[...]
    o_ref[...] = jax.nn.sigmoid(t) * _SCALE + _SHIFT


def _mlp(u_embt, m_embt, w1at, w1bt, b1, w2, b2):
    grid = (_B // _BLK,)
    return pl.pallas_call(
        _mlp_body,
        grid=grid,
        in_specs=[
            pl.BlockSpec((_F, _BLK), lambda i: (0, i)),
            pl.BlockSpec((_F, _BLK), lambda i: (0, i)),
            pl.BlockSpec((_HID, _F), lambda i: (0, 0)),
            pl.BlockSpec((_HID, _F), lambda i: (0, 0)),
            pl.BlockSpec((_HID, 1), lambda i: (0, 0)),
            pl.BlockSpec((_HID, 1), lambda i: (0, 0)),
            pl.BlockSpec((1, 1), lambda i: (0, 0)),
        ],
        out_specs=pl.BlockSpec((1, _BLK), lambda i: (0, i)),
        out_shape=jax.ShapeDtypeStruct((1, _B), jnp.float32),
    )(u_embt, m_embt, w1at, w1bt, b1, w2, b2)


def kernel(user, movie, u_table, m_table, W1, b1, W2, b2):
    u_rows, m_rows = _retile(u_table.T, m_table.T)
    u_idx = user.astype(jnp.int32).reshape(_NW * _NCHUNK, _CHUNK)
    m_idx = movie.astype(jnp.int32).reshape(_NW * _NCHUNK, _CHUNK)
    u_embt, m_embt = _gather(u_idx, m_idx, u_rows, m_rows)
    out = _mlp(u_embt, m_embt, W1[:_F].T, W1[_F:].T,
               b1.reshape(_HID, 1), W2.reshape(_HID, 1), b2.reshape(1, 1))
    return out.reshape(_B, 1)


# speedup vs baseline: 1.9254x; 1.0082x over previous
import functools

import jax
import jax.numpy as jnp
from jax import lax
from jax.experimental import pallas as pl
from jax.experimental.pallas import tpu as pltpu
from jax.experimental.pallas import tpu_sc as plsc

_B = 16384
_F = 16
_HID = 64
_NW = 32
_ROWS_PER_W = _B // _NW
_CHUNK = 128
_NCHUNK = _ROWS_PER_W // _CHUNK
_L = 16

_NU = 1000000
_NM = 100000
_CU = _NU // 128
_CM = _NM // 128
_RU = _NU // 8
_RM_PAD = 12504

_SCALE = 5.0 - 0.5 + 1.0
_SHIFT = 0.5 - 0.5


def _permute_block(in_blk, out_blk):
    lanes = jax.lax.iota(jnp.int32, _L)

    @plsc.parallel_loop(0, 128, unroll=8)
    def _(i):
        r = i >> 3
        s = i & 7
        out_blk[r, pl.ds(s * _L, _L)] = plsc.load_gather(
            in_blk, [lanes, lanes * 0 + i])


def _retile_table(tabt, rows_out, nfull, wid, ring, obuf, in_sem, out_sem):
    n_w = (nfull - wid + _NW - 1) // _NW

    def fire_in(t, slot):
        c = (wid + _NW * t) * 128
        pltpu.async_copy(tabt.at[:, pl.ds(c, 128)], ring.at[slot], in_sem)

    for t in range(8):
        fire_in(t, t)

    def body(t, _):
        slot = lax.rem(t, 8)
        ob = lax.rem(t, 8)
        pltpu.make_async_copy(tabt.at[:, pl.ds(0, 128)],
                              ring.at[slot], in_sem).wait()

        @pl.when(t >= 8)
        def _():
            pltpu.make_async_copy(obuf.at[ob],
                                  rows_out.at[pl.ds(0, 16), :],
                                  out_sem).wait()
        _permute_block(ring.at[slot], obuf.at[ob])
        c16 = (wid + _NW * t) * 16
        pltpu.async_copy(obuf.at[ob], rows_out.at[pl.ds(c16, 16), :], out_sem)

        @pl.when(t + 8 < n_w)
        def _():
            fire_in(t + 8, slot)
        return 0

    jax.lax.fori_loop(0, n_w, body, 0)
    for _ in range(8):
        pltpu.make_async_copy(obuf.at[0], rows_out.at[pl.ds(0, 16), :],
                              out_sem).wait()


def _retile_body(u_tabt, m_tabt, u_rows, m_rows,
                 ring, obuf, rem_in_u, rem_in_m, rem_out, in_sem, out_sem):
    wid = lax.axis_index("s") * 2 + lax.axis_index("c")
    _retile_table(u_tabt, u_rows, _CU, wid, ring, obuf, in_sem, out_sem)
    _retile_table(m_tabt, m_rows, _CM, wid, ring, obuf, in_sem, out_sem)

    lanes = jax.lax.iota(jnp.int32, _L)

    @pl.when(wid == 0)
    def _():
        pltpu.sync_copy(u_tabt.at[:, pl.ds(_CU * 128, 64)], rem_in_u)
        for r in range(8):
            for s in range(8):
                rem_out[r, pl.ds(s * _L, _L)] = plsc.load_gather(
                    rem_in_u, [lanes, jnp.full((_L,), r * 8 + s, jnp.int32)])
        pltpu.sync_copy(rem_out.at[pl.ds(0, 8), :],
                        u_rows.at[pl.ds(_CU * 16, 8), :])

    @pl.when(wid == 1)
    def _():
        pltpu.sync_copy(m_tabt.at[:, pl.ds(_CM * 128, 32)], rem_in_m)
        for r in range(8):
            for s in range(8):
                l = min(r * 8 + s, 31)
                rem_out[r, pl.ds(s * _L, _L)] = plsc.load_gather(
                    rem_in_m, [lanes, jnp.full((_L,), l, jnp.int32)])
        pltpu.sync_copy(rem_out.at[pl.ds(0, 8), :],
                        m_rows.at[pl.ds(_CM * 16, 8), :])


@functools.partial(
    pl.kernel,
    out_type=(
        jax.ShapeDtypeStruct((_RU, 128), jnp.float32),
        jax.ShapeDtypeStruct((_RM_PAD, 128), jnp.float32),
    ),
    mesh=plsc.VectorSubcoreMesh(core_axis_name="c", subcore_axis_name="s"),
    compiler_params=pltpu.CompilerParams(needs_layout_passes=False),
    scratch_types=[
        pltpu.VMEM((8, _F, 128), jnp.float32),
        pltpu.VMEM((8, _F, 128), jnp.float32),
        pltpu.VMEM((_F, 64), jnp.float32),
        pltpu.VMEM((_F, 32), jnp.float32),
        pltpu.VMEM((8, 128), jnp.float32),
        pltpu.SemaphoreType.DMA,
        pltpu.SemaphoreType.DMA,
    ],
)
def _retile(u_tabt, m_tabt, u_rows, m_rows,
            ring, obuf, rem_in_u, rem_in_m, rem_out, in_sem, out_sem):
    _retile_body(u_tabt, m_tabt, u_rows, m_rows,
                 ring, obuf, rem_in_u, rem_in_m, rem_out, in_sem, out_sem)


def _compute_tile_indices(idx_v, tidx_v):
    for j in range(_NCHUNK):
        def body(g, _, j=j):
            iv = idx_v[j, pl.ds(g * _L, _L)]
            tidx_v[j, pl.ds(g * _L, _L)] = jax.lax.shift_right_logical(iv, 3)
            return 0
        jax.lax.fori_loop(0, _CHUNK // _L, body, 0)


def _extract_chunk(idx_v, j, gat, stage):
    lanes = jax.lax.iota(jnp.int32, _L)

    def body(g, _):
        iv = idx_v[j, pl.ds(g * _L, _L)]
        sub = (iv & 7) * _F
        rows = g * _L + lanes
        off = j * _CHUNK + g * _L

        @plsc.parallel_loop(0, _F, unroll=8)
        def _(c):
            stage[c, pl.ds(off, _L)] = plsc.load_gather(gat, [rows, sub + c])
        return 0

    jax.lax.fori_loop(0, _CHUNK // _L, body, 0)


def _gather_body(u_idx, m_idx, u_tab, m_tab, u_out, m_out,
                 u_idx_v, m_idx_v, u_tidx, m_tidx, gat, u_stage, m_stage, sem):
    wid = lax.axis_index("s") * 2 + lax.axis_index("c")
    base = wid * _ROWS_PER_W
    pltpu.sync_copy(u_idx.at[pl.ds(wid * _NCHUNK, _NCHUNK)], u_idx_v)
    pltpu.sync_copy(m_idx.at[pl.ds(wid * _NCHUNK, _NCHUNK)], m_idx_v)
    _compute_tile_indices(u_idx_v, u_tidx)
    _compute_tile_indices(m_idx_v, m_tidx)

    tasks = [(u_tab, u_tidx, u_idx_v, u_stage, j) for j in range(_NCHUNK)]
    tasks += [(m_tab, m_tidx, m_idx_v, m_stage, j) for j in range(_NCHUNK)]

    def fire(t, buf):
        tab, tidx, _, _, j = tasks[t]
        return pltpu.async_copy(tab.at[tidx.at[j]], gat.at[buf], sem)

    handles = {0: fire(0, 0)}
    for t in range(len(tasks)):
        handles[t].wait()
        if t + 1 < len(tasks):
            handles[t + 1] = fire(t + 1, (t + 1) % 2)
        _, _, idx_v, stage, j = tasks[t]
        _extract_chunk(idx_v, j, gat.at[t % 2], stage)

    pltpu.sync_copy(u_stage, u_out.at[:, pl.ds(base, _ROWS_PER_W)])
    pltpu.sync_copy(m_stage, m_out.at[:, pl.ds(base, _ROWS_PER_W)])


@functools.partial(
    pl.kernel,
    out_type=(
        jax.ShapeDtypeStruct((_F, _B), jnp.float32),
        jax.ShapeDtypeStruct((_F, _B), jnp.float32),
    ),
    mesh=plsc.VectorSubcoreMesh(core_axis_name="c", subcore_axis_name="s"),
    compiler_params=pltpu.CompilerParams(needs_layout_passes=False),
    scratch_types=[
        pltpu.VMEM((_NCHUNK, _CHUNK), jnp.int32),
        pltpu.VMEM((_NCHUNK, _CHUNK), jnp.int32),
        pltpu.VMEM((_NCHUNK, _CHUNK), jnp.int32),
        pltpu.VMEM((_NCHUNK, _CHUNK), jnp.int32),
        pltpu.VMEM((2, _CHUNK, 128), jnp.float32),
        pltpu.VMEM((_F, _ROWS_PER_W), jnp.float32),
        pltpu.VMEM((_F, _ROWS_PER_W), jnp.float32),
        pltpu.SemaphoreType.DMA,
    ],
)
def _gather(u_idx, m_idx, u_tab, m_tab, u_out, m_out,
            u_idx_v, m_idx_v, u_tidx, m_tidx, gat, u_stage, m_stage, sem):
    _gather_body(u_idx, m_idx, u_tab, m_tab, u_out, m_out,
                 u_idx_v, m_idx_v, u_tidx, m_tidx, gat, u_stage, m_stage, sem)


_BLK = 2048


def _mlp_body(u_ref, m_ref, w1a_ref, w1b_ref, b1_ref, w2_ref, b2_ref, o_ref):
    h = jnp.dot(w1a_ref[...], u_ref[...],
                preferred_element_type=jnp.float32,
                precision=lax.Precision.HIGHEST)
    h = h + jnp.dot(w1b_ref[...], m_ref[...],
                    preferred_element_type=jnp.float32,
                    precision=lax.Precision.HIGHEST)
    h = jnp.maximum(h + b1_ref[...], 0.0)
    t = jnp.sum(h * w2_ref[...], axis=0, keepdims=True) + b2_ref[...]
    o_ref[...] = jax.nn.sigmoid(t) * _SCALE + _SHIFT


def _mlp(u_embt, m_embt, w1at, w1bt, b1, w2, b2):
    grid = (_B // _BLK,)
    return pl.pallas_call(
        _mlp_body,
        grid=grid,
        in_specs=[
            pl.BlockSpec((_F, _BLK), lambda i: (0, i)),
            pl.BlockSpec((_F, _BLK), lambda i: (0, i)),
            pl.BlockSpec((_HID, _F), lambda i: (0, 0)),
            pl.BlockSpec((_HID, _F), lambda i: (0, 0)),
            pl.BlockSpec((_HID, 1), lambda i: (0, 0)),
            pl.BlockSpec((_HID, 1), lambda i: (0, 0)),
            pl.BlockSpec((1, 1), lambda i: (0, 0)),
        ],
        out_specs=pl.BlockSpec((1, _BLK), lambda i: (0, i)),
        out_shape=jax.ShapeDtypeStruct((1, _B), jnp.float32),
    )(u_embt, m_embt, w1at, w1bt, b1, w2, b2)


def kernel(user, movie, u_table, m_table, W1, b1, W2, b2):
    u_rows, m_rows = _retile(u_table.T, m_table.T)
    u_idx = user.astype(jnp.int32).reshape(_NW * _NCHUNK, _CHUNK)
    m_idx = movie.astype(jnp.int32).reshape(_NW * _NCHUNK, _CHUNK)
    u_embt, m_embt = _gather(u_idx, m_idx, u_rows, m_rows)
    out = _mlp(u_embt, m_embt, W1[:_F].T, W1[_F:].T,
               b1.reshape(_HID, 1), W2.reshape(_HID, 1), b2.reshape(1, 1))
    return out.reshape(_B, 1)
